# baseline (speedup 1.0000x reference)
import functools
import math

import jax
import jax.numpy as jnp
from jax import lax
from jax.experimental import pallas as pl
from jax.experimental.pallas import tpu as pltpu
from jax.experimental.pallas import tpu_sc as plsc

N = 50000
E = 1600000
D_IN = 12
H = 48
EPS = 1e-5
INV_SQRT = 1.0 / math.sqrt(1.0 + EPS)

NC = 2
NS = 16
GSZ = 512
NSET = 4
NPAD = 50176
ROW_PT = NPAD // NS
E_PAD = 1638400
EDGES_PT_A = E_PAD // (NC * NS)
EDGES_PT_C = E_PAD // NS
ZCH = 64


def _zero_fill(zref, ncols):
    z16 = jnp.zeros((16,), jnp.float32)
    for i in range(ZCH):
        for c0 in range(0, ncols, 16):
            zref[i, pl.ds(min(c0, ncols - 16), 16)] = z16


def _make_seg_kernel(ncols, nsrc):
    mesh = plsc.VectorSubcoreMesh(core_axis_name="c", subcore_axis_name="s")
    edges_pt = EDGES_PT_A if nsrc == 1 else EDGES_PT_C
    nunits = edges_pt // GSZ

    @functools.partial(
        pl.kernel,
        mesh=mesh,
        out_type=jax.ShapeDtypeStruct((2 * NPAD, ncols), jnp.float32),
        compiler_params=pltpu.CompilerParams(use_tc_tiling_on_sc=False),
        scratch_types=(
            [pltpu.VMEM_SHARED((NPAD, ncols), jnp.float32)]
            + [pltpu.VMEM((GSZ,), jnp.int32)] * (2 * NSET)
            + [pltpu.VMEM((GSZ, ncols), jnp.float32)] * NSET
            + [pltpu.VMEM((ZCH, ncols), jnp.float32)]
            + [pltpu.SemaphoreType.DMA] * (4 * NSET)
        ),
    )
    def k(*refs):
        tabs = refs[:nsrc]
        src_hbm, dst_hbm, out_hbm, acc = refs[nsrc:nsrc + 4]
        b = nsrc + 4
        src_v = refs[b:b + NSET]
        dst_v = refs[b + NSET:b + 2 * NSET]
        rows = refs[b + 2 * NSET:b + 3 * NSET]
        zero_v = refs[b + 3 * NSET]
        b = b + 3 * NSET + 1
        issem = refs[b:b + NSET]
        idsem = refs[b + NSET:b + 2 * NSET]
        gsem = refs[b + 2 * NSET:b + 3 * NSET]
        ssem = refs[b + 3 * NSET:b + 4 * NSET]
        c = lax.axis_index("c")
        s = lax.axis_index("s")
        _zero_fill(zero_v, ncols)
        rbase = s * ROW_PT

        def zchunk(kk, carry):
            pltpu.sync_copy(zero_v, acc.at[pl.ds(rbase + kk * ZCH, ZCH)])
            return carry

        lax.fori_loop(0, ROW_PT // ZCH, zchunk, 0)
        plsc.subcore_barrier()

        tbase = ((c * NS + s) if nsrc == 1 else s) * edges_pt

        def drain_s(i):
            pltpu.make_async_copy(tabs[0].at[pl.ds(0, GSZ)], rows[i],
                                  ssem[i]).wait()

        def drain_g(i):
            pltpu.make_async_copy(tabs[0].at[pl.ds(0, GSZ)], rows[i],
                                  gsem[i]).wait()

        def wait_is(i):
            pltpu.make_async_copy(src_hbm.at[pl.ds(0, GSZ)], src_v[i],
                                  issem[i]).wait()

        def wait_id(i):
            pltpu.make_async_copy(src_hbm.at[pl.ds(0, GSZ)], dst_v[i],
                                  idsem[i]).wait()

        def issue_gather(i):
            if nsrc == 1:
                pltpu.async_copy(tabs[0].at[src_v[i]], rows[i], gsem[i])
                return

            @pl.when(c == 0)
            def _():
                pltpu.async_copy(tabs[0].at[src_v[i]], rows[i], gsem[i])

            @pl.when(c == 1)
            def _():
                pltpu.async_copy(tabs[1].at[src_v[i]], rows[i], gsem[i])

        def phase(u, j, drain, gather, scatter, idx=True):
            if drain:
                drain_s(j)
            if idx:
                e0 = tbase + u * GSZ
                pltpu.async_copy(src_hbm.at[pl.ds(e0, GSZ)], src_v[j],
                                 issem[j])
                pltpu.async_copy(dst_hbm.at[pl.ds(e0, GSZ)], dst_v[j],
                                 idsem[j])
            if gather:
                i = (j + 3) % NSET
                wait_is(i)
                issue_gather(i)
            if scatter:
                i = (j + 2) % NSET
                drain_g(i)
                wait_id(i)
                pltpu.async_copy(rows[i], acc.at[dst_v[i]], ssem[i],
                                 add=True)

        phase(0, 0, False, False, False)
        phase(1, 1, False, True, False)
        phase(2, 2, False, True, True)
        phase(3, 3, False, True, True)

        def body(kk, carry):
            base = kk * NSET
            for j in range(NSET):
                phase(base + j, j, True, True, True)
            return carry

        lax.fori_loop(1, nunits // NSET, body, 0)
        phase(nunits, 0, True, True, True, idx=False)
        phase(nunits + 1, 1, True, False, True, idx=False)
        drain_s(2)
        drain_s(3)

        plsc.subcore_barrier()
        pltpu.sync_copy(acc.at[pl.ds(rbase, ROW_PT)],
                        out_hbm.at[pl.ds(c * NPAD + rbase, ROW_PT)])

    return k


_seg_a = _make_seg_kernel(16, 1)
_seg_c = _make_seg_kernel(24, 2)


BLK = 3584
GRID = NPAD // BLK


def _dense0(p3, xa, wl, wr, b, g, be):
    def body(p_ref, x_ref, wl_ref, wr_ref, b_ref, g_ref, be_ref,
             h0_ref, h1_ref):
        p = p_ref[0] + p_ref[1]
        r = 1.0 / jnp.maximum(p[:, 12:13], 1.0)
        h = (jnp.dot(p * r, wl_ref[...], preferred_element_type=jnp.float32)
             + jnp.dot(x_ref[...], wr_ref[...],
                       preferred_element_type=jnp.float32)
             + b_ref[...])
        h = jnp.maximum(h * (g_ref[...] * INV_SQRT) + be_ref[...], 0.0)
        h0_ref[...] = h[:, :24]
        h1_ref[...] = h[:, 24:48]

    full = lambda shp: pl.BlockSpec(shp, lambda i: (0,) * len(shp))
    return pl.pallas_call(
        body,
        grid=(GRID,),
        in_specs=[
            pl.BlockSpec((2, BLK, 16), lambda i: (0, i, 0)),
            pl.BlockSpec((BLK, 16), lambda i: (i, 0)),
            full((16, H)), full((16, H)),
            full((1, H)), full((1, H)), full((1, H)),
        ],
        out_specs=[pl.BlockSpec((BLK, 24), lambda i: (i, 0))] * 2,
        out_shape=[jax.ShapeDtypeStruct((NPAD, 24), jnp.float32)] * 2,
    )(p3, xa, wl, wr, b, g, be)


def _dense1a(h0, h1, wra, wrb, b):
    def body(h0_ref, h1_ref, wra_ref, wrb_ref, b_ref, o_ref):
        o_ref[...] = (jnp.dot(h0_ref[...], wra_ref[...],
                              preferred_element_type=jnp.float32)
                      + jnp.dot(h1_ref[...], wrb_ref[...],
                                preferred_element_type=jnp.float32)
                      + b_ref[...])

    full = lambda shp: pl.BlockSpec(shp, lambda i: (0,) * len(shp))
    return pl.pallas_call(
        body,
        grid=(GRID,),
        in_specs=[
            pl.BlockSpec((BLK, 24), lambda i: (i, 0)),
            pl.BlockSpec((BLK, 24), lambda i: (i, 0)),
            full((24, H)), full((24, H)), full((1, H)),
        ],
        out_specs=pl.BlockSpec((BLK, H), lambda i: (i, 0)),
        out_shape=jax.ShapeDtypeStruct((NPAD, H), jnp.float32),
    )(h0, h1, wra, wrb, b)


BLK1 = 2000
GRID1 = N // BLK1


def _dense1b(s3, p3, xr, wla, wlb, g, be):
    def body(s_ref, p_ref, xr_ref, wla_ref, wlb_ref, g_ref, be_ref, o_ref):
        p = p_ref[0] + p_ref[1]
        r = 1.0 / jnp.maximum(p[:, 12:13], 1.0)
        o = (jnp.dot(s_ref[0] * r, wla_ref[...],
                     preferred_element_type=jnp.float32)
             + jnp.dot(s_ref[1] * r, wlb_ref[...],
                       preferred_element_type=jnp.float32)
             + xr_ref[...])
        o_ref[...] = jnp.maximum(o * (g_ref[...] * INV_SQRT) + be_ref[...],
                                 0.0)

    full = lambda shp: pl.BlockSpec(shp, lambda i: (0,) * len(shp))
    return pl.pallas_call(
        body,
        grid=(GRID1,),
        in_specs=[
            pl.BlockSpec((2, BLK1, 24), lambda i: (0, i, 0)),
            pl.BlockSpec((2, BLK1, 16), lambda i: (0, i, 0)),
            pl.BlockSpec((BLK1, H), lambda i: (i, 0)),
            full((24, H)), full((24, H)), full((1, H)), full((1, H)),
        ],
        out_specs=pl.BlockSpec((BLK1, H), lambda i: (i, 0)),
        out_shape=jax.ShapeDtypeStruct((N, H), jnp.float32),
    )(s3, p3, xr, wla, wlb, g, be)


def kernel(x, edge_index, W_l0, b_l0, W_r0, g0, be0, W_l1, b_l1, W_r1, g1,
           be1):
    f32 = jnp.float32
    src = edge_index[0]
    dst = edge_index[1]
    pad_e = E_PAD - E
    src1d = jnp.concatenate([src, jnp.zeros((pad_e,), jnp.int32)])
    trash = N + (jnp.arange(pad_e, dtype=jnp.int32) % (NPAD - N))
    dst1d = jnp.concatenate([dst, trash])
    xa = jnp.zeros((NPAD, 16), f32).at[:N, :D_IN].set(x).at[:N, D_IN].set(1.0)

    p_flat = _seg_a(xa, src1d, dst1d)
    p3 = p_flat.reshape(2, NPAD, 16)

    wl0 = jnp.zeros((16, H), f32).at[:D_IN].set(W_l0)
    wr0 = jnp.zeros((16, H), f32).at[:D_IN].set(W_r0)
    h0, h1 = _dense0(p3, xa, wl0, wr0, b_l0.reshape(1, H),
                     g0.reshape(1, H), be0.reshape(1, H))

    xr = _dense1a(h0, h1, W_r1[:24], W_r1[24:], b_l1.reshape(1, H))
    s_flat = _seg_c(h0, h1, src1d, dst1d)
    s3 = s_flat.reshape(2, NPAD, 24)

    return _dense1b(s3, p3, xr, W_l1[:24], W_l1[24:],
                    g1.reshape(1, H), be1.reshape(1, H))


# speedup vs baseline: 1.0984x; 1.0984x over previous
import functools
import math

import jax
import jax.numpy as jnp
from jax import lax
from jax.experimental import pallas as pl
from jax.experimental.pallas import tpu as pltpu
from jax.experimental.pallas import tpu_sc as plsc

N = 50000
E = 1600000
D_IN = 12
H = 48
EPS = 1e-5
INV_SQRT = 1.0 / math.sqrt(1.0 + EPS)

NC = 2
NS = 16
GSZ = 512
NSET = 4
NPAD = 50176
ROW_PT = NPAD // NS
E_PAD = 1638400
EDGES_PT_A = E_PAD // (NC * NS)
EDGES_PT_C = E_PAD // NS
ZCH = 64


def _zero_fill(zref, ncols):
    z16 = jnp.zeros((16,), jnp.float32)
    for i in range(ZCH):
        for c0 in range(0, ncols, 16):
            zref[i, pl.ds(min(c0, ncols - 16), 16)] = z16


def _make_seg_kernel(ncols, nsrc):
    mesh = plsc.VectorSubcoreMesh(core_axis_name="c", subcore_axis_name="s")
    edges_pt = EDGES_PT_A if nsrc == 1 else EDGES_PT_C
    nunits = edges_pt // GSZ

    @functools.partial(
        pl.kernel,
        mesh=mesh,
        out_type=jax.ShapeDtypeStruct((2 * NPAD, ncols), jnp.float32),
        compiler_params=pltpu.CompilerParams(use_tc_tiling_on_sc=False),
        scratch_types=(
            [pltpu.VMEM_SHARED((NPAD, ncols), jnp.float32)]
            + [pltpu.VMEM((GSZ,), jnp.int32)] * (2 * NSET)
            + [pltpu.VMEM((GSZ, ncols), jnp.float32)] * NSET
            + [pltpu.VMEM((ZCH, ncols), jnp.float32)]
            + [pltpu.SemaphoreType.DMA] * (4 * NSET)
        ),
    )
    def k(*refs):
        tabs = refs[:nsrc]
        src_hbm, dst_hbm, out_hbm, acc = refs[nsrc:nsrc + 4]
        b = nsrc + 4
        src_v = refs[b:b + NSET]
        dst_v = refs[b + NSET:b + 2 * NSET]
        rows = refs[b + 2 * NSET:b + 3 * NSET]
        zero_v = refs[b + 3 * NSET]
        b = b + 3 * NSET + 1
        issem = refs[b:b + NSET]
        idsem = refs[b + NSET:b + 2 * NSET]
        gsem = refs[b + 2 * NSET:b + 3 * NSET]
        ssem = refs[b + 3 * NSET:b + 4 * NSET]
        c = lax.axis_index("c")
        s = lax.axis_index("s")
        _zero_fill(zero_v, ncols)
        rbase = s * ROW_PT

        def zchunk(kk, carry):
            pltpu.sync_copy(zero_v, acc.at[pl.ds(rbase + kk * ZCH, ZCH)])
            return carry

        lax.fori_loop(0, ROW_PT // ZCH, zchunk, 0)
        plsc.subcore_barrier()

        tbase = ((c * NS + s) if nsrc == 1 else s) * edges_pt

        def drain_s(i):
            pltpu.make_async_copy(tabs[0].at[pl.ds(0, GSZ)], rows[i],
                                  ssem[i]).wait()

        def drain_g(i):
            pltpu.make_async_copy(tabs[0].at[pl.ds(0, GSZ)], rows[i],
                                  gsem[i]).wait()

        def wait_is(i):
            pltpu.make_async_copy(src_hbm.at[pl.ds(0, GSZ)], src_v[i],
                                  issem[i]).wait()

        def wait_id(i):
            pltpu.make_async_copy(src_hbm.at[pl.ds(0, GSZ)], dst_v[i],
                                  idsem[i]).wait()

        def issue_gather(i):
            if nsrc == 1:
                pltpu.async_copy(tabs[0].at[src_v[i]], rows[i], gsem[i])
                return

            @pl.when(c == 0)
            def _():
                pltpu.async_copy(tabs[0].at[src_v[i]], rows[i], gsem[i])

            @pl.when(c == 1)
            def _():
                pltpu.async_copy(tabs[1].at[src_v[i]], rows[i], gsem[i])

        def phase(u, j, drain, gather, scatter, idx=True):
            if drain:
                drain_s(j)
            if idx:
                e0 = tbase + u * GSZ
                pltpu.async_copy(src_hbm.at[pl.ds(e0, GSZ)], src_v[j],
                                 issem[j])
                pltpu.async_copy(dst_hbm.at[pl.ds(e0, GSZ)], dst_v[j],
                                 idsem[j])
            if gather:
                i = (j + 3) % NSET
                wait_is(i)
                issue_gather(i)
            if scatter:
                i = (j + 2) % NSET
                drain_g(i)
                wait_id(i)
                pltpu.async_copy(rows[i], acc.at[dst_v[i]], ssem[i],
                                 add=True)

        phase(0, 0, False, False, False)
        phase(1, 1, False, True, False)
        phase(2, 2, False, True, True)
        phase(3, 3, False, True, True)

        def body(kk, carry):
            base = kk * NSET
            for j in range(NSET):
                phase(base + j, j, True, True, True)
            return carry

        lax.fori_loop(1, nunits // NSET, body, 0)
        phase(nunits, 0, True, True, True, idx=False)
        phase(nunits + 1, 1, True, False, True, idx=False)
        drain_s(2)
        drain_s(3)

        plsc.subcore_barrier()
        pltpu.sync_copy(acc.at[pl.ds(rbase, ROW_PT)],
                        out_hbm.at[pl.ds(c * NPAD + rbase, ROW_PT)])

    return k


_seg_a = _make_seg_kernel(16, 1)
_seg_c = _make_seg_kernel(24, 2)


BLK = 3584
GRID = NPAD // BLK


def _dense0(p3, xa, wl, wr, b, g, be):
    def body(p_ref, x_ref, wl_ref, wr_ref, b_ref, g_ref, be_ref,
             h0_ref, h1_ref):
        p = p_ref[0] + p_ref[1]
        r = 1.0 / jnp.maximum(p[:, 12:13], 1.0)
        h = (jnp.dot(p * r, wl_ref[...], preferred_element_type=jnp.float32)
             + jnp.dot(x_ref[...], wr_ref[...],
                       preferred_element_type=jnp.float32)
             + b_ref[...])
        h = jnp.maximum(h * (g_ref[...] * INV_SQRT) + be_ref[...], 0.0)
        h0_ref[...] = h[:, :24]
        h1_ref[...] = h[:, 24:48]

    full = lambda shp: pl.BlockSpec(shp, lambda i: (0,) * len(shp))
    return pl.pallas_call(
        body,
        grid=(GRID1,),
        in_specs=[
            pl.BlockSpec((2, BLK1, 16), lambda i: (0, i, 0)),
            pl.BlockSpec((BLK1, 16), lambda i: (i, 0)),
            full((16, H)), full((16, H)),
            full((1, H)), full((1, H)), full((1, H)),
        ],
        out_specs=[pl.BlockSpec((BLK1, 24), lambda i: (i, 0))] * 2,
        out_shape=[jax.ShapeDtypeStruct((N, 24), jnp.float32)] * 2,
    )(p3, xa, wl, wr, b, g, be)


BLK1 = 2000
GRID1 = N // BLK1


def _dense1(s3, p3, h0, h1, wla, wlb, wra, wrb, b, g, be):
    def body(s_ref, p_ref, h0_ref, h1_ref, wla_ref, wlb_ref, wra_ref,
             wrb_ref, b_ref, g_ref, be_ref, o_ref):
        p = p_ref[0] + p_ref[1]
        r = 1.0 / jnp.maximum(p[:, 12:13], 1.0)
        o = (jnp.dot(s_ref[0] * r, wla_ref[...],
                     preferred_element_type=jnp.float32)
             + jnp.dot(s_ref[1] * r, wlb_ref[...],
                       preferred_element_type=jnp.float32)
             + jnp.dot(h0_ref[...], wra_ref[...],
                       preferred_element_type=jnp.float32)
             + jnp.dot(h1_ref[...], wrb_ref[...],
                       preferred_element_type=jnp.float32)
             + b_ref[...])
        o_ref[...] = jnp.maximum(o * (g_ref[...] * INV_SQRT) + be_ref[...],
                                 0.0)

    full = lambda shp: pl.BlockSpec(shp, lambda i: (0,) * len(shp))
    return pl.pallas_call(
        body,
        grid=(GRID1,),
        in_specs=[
            pl.BlockSpec((2, BLK1, 24), lambda i: (0, i, 0)),
            pl.BlockSpec((2, BLK1, 16), lambda i: (0, i, 0)),
            pl.BlockSpec((BLK1, 24), lambda i: (i, 0)),
            pl.BlockSpec((BLK1, 24), lambda i: (i, 0)),
            full((24, H)), full((24, H)), full((24, H)), full((24, H)),
            full((1, H)), full((1, H)), full((1, H)),
        ],
        out_specs=pl.BlockSpec((BLK1, H), lambda i: (i, 0)),
        out_shape=jax.ShapeDtypeStruct((N, H), jnp.float32),
    )(s3, p3, h0, h1, wla, wlb, wra, wrb, b, g, be)


def kernel(x, edge_index, W_l0, b_l0, W_r0, g0, be0, W_l1, b_l1, W_r1, g1,
           be1):
    f32 = jnp.float32
    src = edge_index[0]
    dst = edge_index[1]
    pad_e = E_PAD - E
    src1d = jnp.concatenate([src, jnp.zeros((pad_e,), jnp.int32)])
    trash = N + (jnp.arange(pad_e, dtype=jnp.int32) % (NPAD - N))
    dst1d = jnp.concatenate([dst, trash])
    xa = jnp.concatenate(
        [x, jnp.ones((N, 1), f32), jnp.zeros((N, 3), f32)], axis=1)

    p_flat = _seg_a(xa, src1d, dst1d)
    p3 = p_flat.reshape(2, NPAD, 16)

    wl0 = jnp.zeros((16, H), f32).at[:D_IN].set(W_l0)
    wr0 = jnp.zeros((16, H), f32).at[:D_IN].set(W_r0)
    h0, h1 = _dense0(p3, xa, wl0, wr0, b_l0.reshape(1, H),
                     g0.reshape(1, H), be0.reshape(1, H))

    s_flat = _seg_c(h0, h1, src1d, dst1d)
    s3 = s_flat.reshape(2, NPAD, 24)

    return _dense1(s3, p3, h0, h1, W_l1[:24], W_l1[24:], W_r1[:24],
                   W_r1[24:], b_l1.reshape(1, H), g1.reshape(1, H),
                   be1.reshape(1, H))
